# baseline (speedup 1.0000x reference)
import functools

import jax
import jax.numpy as jnp
import numpy as np
from jax.experimental import pallas as pl
from jax.experimental.pallas import tpu as pltpu
from jax.experimental.shard_map import shard_map
from jax.sharding import Mesh, PartitionSpec as P

_NEG_SLOPE = 0.01
_K = 16


def _leaky(x):
    return jnp.where(x > 0, x, _NEG_SLOPE * x)


def _fused_kernel(xq_ref, xall_ref, pts_ref, ptile_ref, wcat_ref, bcat_ref,
                  wf_ref, bf_ref, o_ref, *, k, nblk1):
    xall = xall_ref[0]
    xq = xq_ref[0]

    d = None
    for a in range(3):
        diff = xall[:, a:a + 1] - xq[a:a + 1, :]
        sq = diff * diff
        d = sq if d is None else d + sq

    neg_inf = jnp.float32(-jnp.inf)
    n_all = d.shape[0]

    def _desc_maxima(dq, count):
        ms = [jnp.max(dq, axis=0, keepdims=True)]
        for _ in range(count - 1):
            ms.append(jnp.max(jnp.where(dq >= ms[-1], neg_inf, dq),
                              axis=0, keepdims=True))
        return ms

    ka = k + 1
    a = _desc_maxima(d[: n_all // 2], ka)
    b = _desc_maxima(d[n_all // 2:], ka)
    cands = [b[ka - 1], a[ka - 1]]
    for i in range(1, ka):
        cands.append(jnp.minimum(a[i - 1], b[ka - 1 - i]))
    tau = cands[0]
    for c in cands[1:]:
        tau = jnp.maximum(tau, c)
    m1 = jnp.maximum(a[0], b[0])
    mask = jnp.where(d >= tau, 1.0, 0.0)
    mask = jnp.where(d == m1, 0.0, mask)

    lp_full = _leaky(pts_ref[0])
    gsum = jnp.dot(lp_full, mask,
                   preferred_element_type=jnp.float32)

    p = ptile_ref[0]
    lp = _leaky(p)
    x0 = jnp.concatenate([lp, gsum], axis=0)
    acc = jnp.dot(wcat_ref[...], x0,
                  preferred_element_type=jnp.float32)
    h = (acc + bcat_ref[...]) * (1.0 / (k + 1.0)) + p

    for blk in range(nblk1):
        lph = _leaky(h)
        acc = jnp.dot(wf_ref[blk], lph,
                      preferred_element_type=jnp.float32)
        h = acc + bf_ref[blk] + h

    o_ref[0] = h.astype(o_ref.dtype)


def _run_chip(xyz, points, w_cat, b_cat, w_f, b_f):
    B, C, N = points.shape
    nblk1 = int(w_f.shape[0])
    if N % 1024 == 0:
        TN = 1024
    elif N % 512 == 0:
        TN = 512
    elif N % 128 == 0:
        TN = 128
    else:
        TN = N
    xyz_nc = jnp.transpose(xyz, (0, 2, 1))

    body = functools.partial(_fused_kernel, k=_K, nblk1=nblk1)
    return pl.pallas_call(
        body,
        out_shape=jax.ShapeDtypeStruct((B, C, N), points.dtype),
        grid=(B, N // TN),
        in_specs=[
            pl.BlockSpec((1, 3, TN), lambda b, n: (b, 0, n)),
            pl.BlockSpec((1, N, 3), lambda b, n: (b, 0, 0)),
            pl.BlockSpec((1, C, N), lambda b, n: (b, 0, 0)),
            pl.BlockSpec((1, C, TN), lambda b, n: (b, 0, n)),
            pl.BlockSpec((C, 2 * C), lambda b, n: (0, 0)),
            pl.BlockSpec((C, 1), lambda b, n: (0, 0)),
            pl.BlockSpec((nblk1, C, C), lambda b, n: (0, 0, 0)),
            pl.BlockSpec((nblk1, C, 1), lambda b, n: (0, 0, 0)),
        ],
        out_specs=pl.BlockSpec((1, C, TN), lambda b, n: (b, 0, n)),
        compiler_params=pltpu.CompilerParams(
            dimension_semantics=("parallel", "arbitrary")),
    )(xyz, xyz_nc, points, points, w_cat, b_cat, w_f, b_f)


def kernel(xyz, points, w_cat, b_cat, w_f, b_f):
    devs = jax.devices()
    B = points.shape[0]
    nd = 1
    for cand in (4, 2):
        if len(devs) >= cand and B % cand == 0:
            nd = cand
            break
    if nd == 1:
        return _run_chip(xyz, points, w_cat, b_cat, w_f, b_f)
    mesh = Mesh(np.asarray(devs[:nd]), ("b",))
    f = shard_map(_run_chip, mesh=mesh,
                  in_specs=(P("b"), P("b"), P(), P(), P(), P()),
                  out_specs=P("b"), check_rep=False)
    return f(xyz, points, w_cat, b_cat, w_f, b_f)


# speedup vs baseline: 2.9725x; 2.9725x over previous
import functools

import jax
import jax.numpy as jnp
from jax.experimental import pallas as pl
from jax.experimental.pallas import tpu as pltpu

_NEG_SLOPE = 0.01
_K = 16


def _leaky(x):
    return jnp.where(x > 0, x, _NEG_SLOPE * x)


def _fused_kernel(xq_ref, xall_ref, pts_ref, ptile_ref, wcat_ref, bcat_ref,
                  wf_ref, bf_ref, o_ref, *, k, nblk1):
    xall = xall_ref[0]
    xq = xq_ref[0]

    nj = jnp.sum(xall * xall, axis=1, keepdims=True)
    ni = jnp.sum(xq * xq, axis=0, keepdims=True)
    ip2 = jnp.dot(xall * -2.0, xq,
                  preferred_element_type=jnp.float32)
    d = (ip2 + nj) + ni

    neg_inf = jnp.float32(-jnp.inf)
    n_all = d.shape[0]

    def _desc_maxima(dq, count):
        ms = [jnp.max(dq, axis=0, keepdims=True)]
        for _ in range(count - 1):
            ms.append(jnp.max(jnp.where(dq >= ms[-1], neg_inf, dq),
                              axis=0, keepdims=True))
        return ms

    ka = k + 1
    a = _desc_maxima(d[: n_all // 2], ka)
    b = _desc_maxima(d[n_all // 2:], ka)
    cands = [b[ka - 1], a[ka - 1]]
    for i in range(1, ka):
        cands.append(jnp.minimum(a[i - 1], b[ka - 1 - i]))
    tau = cands[0]
    for c in cands[1:]:
        tau = jnp.maximum(tau, c)
    m1 = jnp.maximum(a[0], b[0])
    mask = jnp.where(d >= tau, 1.0, 0.0)
    mask = jnp.where(d == m1, 0.0, mask)

    lp_full = _leaky(pts_ref[0])
    gsum = jnp.dot(lp_full, mask,
                   preferred_element_type=jnp.float32)

    p = ptile_ref[0]
    lp = _leaky(p)
    x0 = jnp.concatenate([lp, gsum], axis=0)
    acc = jnp.dot(wcat_ref[...], x0,
                  preferred_element_type=jnp.float32)
    h = (acc + bcat_ref[...]) * (1.0 / (k + 1.0)) + p

    for blk in range(nblk1):
        lph = _leaky(h)
        acc = jnp.dot(wf_ref[blk], lph,
                      preferred_element_type=jnp.float32)
        h = acc + bf_ref[blk] + h

    o_ref[0] = h.astype(o_ref.dtype)


def _run_chip(xyz, points, w_cat, b_cat, w_f, b_f):
    B, C, N = points.shape
    nblk1 = int(w_f.shape[0])
    if N % 1024 == 0:
        TN = 1024
    elif N % 512 == 0:
        TN = 512
    elif N % 128 == 0:
        TN = 128
    else:
        TN = N
    xyz_nc = jnp.transpose(xyz, (0, 2, 1))

    body = functools.partial(_fused_kernel, k=_K, nblk1=nblk1)
    return pl.pallas_call(
        body,
        out_shape=jax.ShapeDtypeStruct((B, C, N), points.dtype),
        grid=(B, N // TN),
        in_specs=[
            pl.BlockSpec((1, 3, TN), lambda b, n: (b, 0, n)),
            pl.BlockSpec((1, N, 3), lambda b, n: (b, 0, 0)),
            pl.BlockSpec((1, C, N), lambda b, n: (b, 0, 0)),
            pl.BlockSpec((1, C, TN), lambda b, n: (b, 0, n)),
            pl.BlockSpec((C, 2 * C), lambda b, n: (0, 0)),
            pl.BlockSpec((C, 1), lambda b, n: (0, 0)),
            pl.BlockSpec((nblk1, C, C), lambda b, n: (0, 0, 0)),
            pl.BlockSpec((nblk1, C, 1), lambda b, n: (0, 0, 0)),
        ],
        out_specs=pl.BlockSpec((1, C, TN), lambda b, n: (b, 0, n)),
        compiler_params=pltpu.CompilerParams(
            dimension_semantics=("parallel", "arbitrary")),
    )(xyz, xyz_nc, points, points, w_cat, b_cat, w_f, b_f)


def kernel(xyz, points, w_cat, b_cat, w_f, b_f):
    return _run_chip(xyz, points, w_cat, b_cat, w_f, b_f)
